# Initial kernel scaffold; baseline (speedup 1.0000x reference)
import functools

import jax
import jax.numpy as jnp
from jax import lax
from jax.experimental import pallas as pl
from jax.experimental.pallas import tpu as pltpu
from jax.experimental.pallas import tpu_sc as plsc

_BATCH = 16384
_LANES = 16

_info = plsc.get_sparse_core_info()
_NC = _info.num_cores
_NS = _info.num_subcores
_NW = _NC * _NS
_BPW = _BATCH // _NW


def _bias_body(u_hbm, i_hbm, ub_hbm, ib_hbm, gb_hbm, out_hbm,
               uidx_v, iidx_v, urows_v, irows_v, gb_s, usem, isem):
    wid = lax.axis_index("s") * _NC + lax.axis_index("c")
    base = wid * _BPW

    pltpu.sync_copy(u_hbm.at[pl.ds(base, _BPW)], uidx_v)
    pltpu.sync_copy(i_hbm.at[pl.ds(base, _BPW)], iidx_v)

    ucopy = pltpu.async_copy(ub_hbm.at[uidx_v], urows_v, usem)
    icopy = pltpu.async_copy(ib_hbm.at[iidx_v], irows_v, isem)

    pltpu.sync_copy(gb_hbm, gb_s)
    g = gb_s[0]

    ucopy.wait()
    icopy.wait()

    for j in range(_BPW // _LANES):
        sl = pl.ds(j * _LANES, _LANES)
        urows_v[sl] = urows_v[sl] + irows_v[sl] + g

    pltpu.sync_copy(urows_v, out_hbm.at[pl.ds(base, _BPW)])


@jax.jit
def _bias_sc(u_id, i_id, ub, ib, gb):
    return pl.kernel(
        _bias_body,
        out_type=jax.ShapeDtypeStruct((_BATCH,), jnp.float32),
        mesh=plsc.VectorSubcoreMesh(core_axis_name="c", subcore_axis_name="s"),
        scratch_types=[
            pltpu.VMEM((_BPW,), jnp.int32),
            pltpu.VMEM((_BPW,), jnp.int32),
            pltpu.VMEM((_BPW,), jnp.float32),
            pltpu.VMEM((_BPW,), jnp.float32),
            pltpu.SMEM((1,), jnp.float32),
            pltpu.SemaphoreType.DMA,
            pltpu.SemaphoreType.DMA,
        ],
    )(u_id, i_id, ub, ib, gb)


def kernel(u_id, i_id, user_bias, item_bias, global_bias):
    return _bias_sc(
        u_id.astype(jnp.int32),
        i_id.astype(jnp.int32),
        user_bias.reshape(-1),
        item_bias.reshape(-1),
        global_bias,
    )


# speedup vs baseline: 1.0729x; 1.0729x over previous
import functools

import jax
import jax.numpy as jnp
from jax import lax
from jax.experimental import pallas as pl
from jax.experimental.pallas import tpu as pltpu
from jax.experimental.pallas import tpu_sc as plsc

_BATCH = 16384
_LANES = 16

_info = plsc.get_sparse_core_info()
_NC = _info.num_cores
_NS = _info.num_subcores
_NW = _NC * _NS
_BPW = _BATCH // _NW


def _bias_body(u_hbm, i_hbm, ub_hbm, ib_hbm, gb_hbm, out_hbm,
               uidx_v, iidx_v, urows_v, irows_v, gb_v, usem, isem):
    wid = lax.axis_index("s") * _NC + lax.axis_index("c")
    base = wid * _BPW

    pltpu.sync_copy(u_hbm.at[pl.ds(base, _BPW)], uidx_v)
    pltpu.sync_copy(i_hbm.at[pl.ds(base, _BPW)], iidx_v)

    ucopy = pltpu.async_copy(ub_hbm.at[uidx_v], urows_v, usem)
    icopy = pltpu.async_copy(ib_hbm.at[iidx_v], irows_v, isem)

    pltpu.sync_copy(gb_hbm, gb_v)
    g = gb_v[...]

    ucopy.wait()
    icopy.wait()

    for j in range(_BPW // _LANES):
        sl = pl.ds(j * _LANES, _LANES)
        urows_v[sl] = urows_v[sl] + irows_v[sl] + g

    pltpu.sync_copy(urows_v, out_hbm.at[pl.ds(base, _BPW)])


@jax.jit
def _bias_sc(u_id, i_id, ub, ib, gb):
    return pl.kernel(
        _bias_body,
        out_type=jax.ShapeDtypeStruct((_BATCH,), jnp.float32),
        mesh=plsc.VectorSubcoreMesh(core_axis_name="c", subcore_axis_name="s"),
        scratch_types=[
            pltpu.VMEM((_BPW,), jnp.int32),
            pltpu.VMEM((_BPW,), jnp.int32),
            pltpu.VMEM((_BPW,), jnp.float32),
            pltpu.VMEM((_BPW,), jnp.float32),
            pltpu.VMEM((_LANES,), jnp.float32),
            pltpu.SemaphoreType.DMA,
            pltpu.SemaphoreType.DMA,
        ],
    )(u_id, i_id, ub, ib, gb)


def kernel(u_id, i_id, user_bias, item_bias, global_bias):
    return _bias_sc(
        u_id.astype(jnp.int32),
        i_id.astype(jnp.int32),
        user_bias.reshape(-1),
        item_bias.reshape(-1),
        jnp.broadcast_to(global_bias, (_LANES,)),
    )
